# baseline (speedup 1.0000x reference)
import functools

import jax
import jax.numpy as jnp
from jax import lax
from jax.experimental import pallas as pl
from jax.experimental.pallas import tpu as pltpu
from jax.experimental.pallas import tpu_sc as plsc

_NUM_CORES = 2
_NUM_SUBCORES = 16
_NUM_WORKERS = _NUM_CORES * _NUM_SUBCORES
_LANES = 16
_CHUNK = 16000


def _sc_sse_body(p_hbm, t_hbm, o_hbm, pb, tb, ob, sem_p0, sem_p1, sem_t0, sem_t1):
    wid = lax.axis_index("s") * _NUM_CORES + lax.axis_index("c")
    n_total = p_hbm.shape[0]
    per_worker = n_total // _NUM_WORKERS
    n_chunks = per_worker // _CHUNK
    base = wid * per_worker

    sem_p = (sem_p0, sem_p1)
    sem_t = (sem_t0, sem_t1)

    def copies(k, slot):
        src = pl.ds(base + k * _CHUNK, _CHUNK)
        cp = pltpu.make_async_copy(p_hbm.at[src], pb.at[slot], sem_p[slot])
        ct = pltpu.make_async_copy(t_hbm.at[src], tb.at[slot], sem_t[slot])
        return cp, ct

    cp, ct = copies(0, 0)
    cp.start()
    ct.start()

    acc = jnp.zeros((_LANES,), jnp.float32)
    for k in range(n_chunks):
        slot = k % 2
        if k + 1 < n_chunks:
            cpn, ctn = copies(k + 1, 1 - slot)
            cpn.start()
            ctn.start()
        cp, ct = copies(k, slot)
        cp.wait()
        ct.wait()

        pb_s = pb.at[slot]
        tb_s = tb.at[slot]

        def body(j, a):
            off = j * _LANES
            d = pb_s[pl.ds(off, _LANES)] - tb_s[pl.ds(off, _LANES)]
            return a + d * d

        acc = lax.fori_loop(0, _CHUNK // _LANES, body, acc, unroll=8)

    ob[...] = acc
    pltpu.sync_copy(ob, o_hbm.at[wid])


def _sc_partial_sums(pred_flat, target_flat):
    mesh = plsc.VectorSubcoreMesh(core_axis_name="c", subcore_axis_name="s")
    kern = functools.partial(
        pl.kernel,
        mesh=mesh,
        out_type=jax.ShapeDtypeStruct((_NUM_WORKERS, _LANES), jnp.float32),
        scratch_types=[
            pltpu.VMEM((2, _CHUNK), jnp.float32),
            pltpu.VMEM((2, _CHUNK), jnp.float32),
            pltpu.VMEM((_LANES,), jnp.float32),
            pltpu.SemaphoreType.DMA,
            pltpu.SemaphoreType.DMA,
            pltpu.SemaphoreType.DMA,
            pltpu.SemaphoreType.DMA,
        ],
    )(_sc_sse_body)
    return kern(pred_flat, target_flat)


def kernel(pred, target, batch_idx, num_graphs):
    del batch_idx
    n_rows, n_feat = pred.shape
    partials = _sc_partial_sums(
        pred.reshape(n_rows * n_feat), target.reshape(n_rows * n_feat)
    )
    return jnp.sum(partials) / num_graphs
